# Initial kernel scaffold; baseline (speedup 1.0000x reference)
import jax
import jax.numpy as jnp
from jax.experimental import pallas as pl


def kernel(x, emb_table, W, b):
    raise NotImplementedError("write your pallas kernel here")


# speedup vs baseline: 2.9657x; 2.9657x over previous
import functools

import jax
import jax.numpy as jnp
from jax import lax
from jax.experimental import pallas as pl
from jax.experimental.pallas import tpu as pltpu
from jax.experimental.pallas import tpu_sc as plsc

_B = 16384
_L = 200
_D = 64
_C = 2

_NC = 2
_NS = 16
_NW = _NC * _NS
_NB = _B // _NW
_G = _D // 16
_SPLIT = 128
_REST = _L - _SPLIT


def _body(table_h, xflat_h, w_h, bp_h, out_h,
          idx_v, rows0_v, rows1_v, out_v, w_v, b_v, sem0, sem1):
  wid = lax.axis_index("s") * _NC + lax.axis_index("c")
  base_b = wid * _NB

  pltpu.sync_copy(xflat_h.at[pl.ds(base_b * _L, _NB * _L)], idx_v)
  pltpu.sync_copy(w_h, w_v)
  pltpu.sync_copy(bp_h, b_v)

  sems = (sem0, sem1)
  rows0 = (rows0_v.at[0], rows0_v.at[1])
  rows1 = (rows1_v.at[0], rows1_v.at[1])

  w_regs = [[w_v[c, pl.ds(g * 16, 16)] for g in range(_G)] for c in range(_C)]
  bp_reg = b_v[...]
  lane = lax.iota(jnp.int32, 16)
  zero = jnp.zeros((16,), jnp.float32)
  neg = jnp.full((16,), -jnp.inf, dtype=jnp.float32)

  def issue(b, s):
    off = pl.multiple_of(b * _L, 8)
    pltpu.async_copy(table_h.at[idx_v.at[pl.ds(off, _SPLIT)]], rows0[s], sems[s])
    pltpu.async_copy(table_h.at[idx_v.at[pl.ds(off + _SPLIT, _REST)]],
                     rows1[s], sems[s])

  def wait(s):
    pltpu.make_async_copy(table_h.at[pl.ds(0, _SPLIT)], rows0[s], sems[s]).wait()
    pltpu.make_async_copy(table_h.at[pl.ds(0, _REST)], rows1[s], sems[s]).wait()

  def compute(s, j, v):
    def red0(r, accs):
      return tuple(
          jnp.maximum(accs[g], rows0[s][r, pl.ds(g * 16, 16)])
          for g in range(_G))

    def red1(r, accs):
      return tuple(
          jnp.maximum(accs[g], rows1[s][r, pl.ds(g * 16, 16)])
          for g in range(_G))

    accs = lax.fori_loop(0, _SPLIT, red0, (neg,) * _G)
    accs = lax.fori_loop(0, _REST, red1, accs)

    for c in range(_C):
      p = accs[0] * w_regs[c][0]
      for g in range(1, _G):
        p = p + accs[g] * w_regs[c][g]
      v = jnp.where(lane == (_C * j + c), jnp.sum(p), v)
    return v

  issue(0, 0)
  issue(1, 1)

  @pl.loop(0, _NB, step=8)
  def _(g):
    v = zero
    for j in range(8):
      b = g + j
      s = j % 2
      wait(s)
      v = compute(s, j, v)
      b2 = b + 2

      @pl.when(b2 < _NB)
      def _():
        issue(b2, s)

    out_v[pl.ds(pl.multiple_of(g * _C, 16), 16)] = v + bp_reg

  pltpu.sync_copy(out_v, out_h.at[pl.ds(base_b * _C, _NB * _C)])


def kernel(x, emb_table, W, b):
  xflat = x.reshape(-1).astype(jnp.int32)
  btiled = jnp.tile(b.astype(jnp.float32), 16 // _C)

  run = pl.kernel(
      _body,
      out_type=jax.ShapeDtypeStruct((_B * _C,), jnp.float32),
      mesh=plsc.VectorSubcoreMesh(core_axis_name="c", subcore_axis_name="s"),
      compiler_params=pltpu.CompilerParams(
          needs_layout_passes=False, use_tc_tiling_on_sc=False),
      scratch_types=[
          pltpu.VMEM((_NB * _L,), jnp.int32),
          pltpu.VMEM((2, _SPLIT, _D), jnp.float32),
          pltpu.VMEM((2, _REST, _D), jnp.float32),
          pltpu.VMEM((_NB * _C,), jnp.float32),
          pltpu.VMEM((_C, _D), jnp.float32),
          pltpu.VMEM((16,), jnp.float32),
          pltpu.SemaphoreType.DMA,
          pltpu.SemaphoreType.DMA,
      ],
  )
  return run(emb_table, xflat, W, btiled).reshape(_B, _C)


# speedup vs baseline: 3.1086x; 1.0482x over previous
import functools

import jax
import jax.numpy as jnp
from jax import lax
from jax.experimental import pallas as pl
from jax.experimental.pallas import tpu as pltpu
from jax.experimental.pallas import tpu_sc as plsc

_B = 16384
_L = 200
_D = 64
_C = 2

_NC = 2
_NS = 16
_NW = _NC * _NS
_NB = _B // _NW
_G = _D // 16
_SPLIT = 128
_REST = _L - _SPLIT


def _body(table_h, xflat_h, w_h, bp_h, out_h,
          idx_v, rows_v, out_v, w_v, b_v, sem0, sem1):
  wid = lax.axis_index("s") * _NC + lax.axis_index("c")
  base_b = wid * _NB

  pltpu.sync_copy(xflat_h.at[pl.ds(base_b * _L, _NB * _L)], idx_v)
  pltpu.sync_copy(w_h, w_v)
  pltpu.sync_copy(bp_h, b_v)

  sems = (sem0, sem1)
  rows = (rows_v.at[0], rows_v.at[1])

  w_regs = [[w_v[c, pl.ds(g * 16, 16)] for g in range(_G)] for c in range(_C)]
  bp_reg = b_v[...]
  lane = lax.iota(jnp.int32, 16)
  zero = jnp.zeros((16,), jnp.float32)
  neg = jnp.full((16,), -jnp.inf, dtype=jnp.float32)

  def issue(b, s):
    off = pl.multiple_of(b * _L, 8)
    pltpu.async_copy(table_h.at[idx_v.at[pl.ds(off, _SPLIT)]],
                     rows[s].at[pl.ds(0, _SPLIT)], sems[s])
    pltpu.async_copy(table_h.at[idx_v.at[pl.ds(off + _SPLIT, _REST)]],
                     rows[s].at[pl.ds(_SPLIT, _REST)], sems[s])

  def wait(s):
    pltpu.make_async_copy(table_h.at[pl.ds(0, _L)], rows[s], sems[s]).wait()

  def compute(s, j, v):
    def red(r, accs):
      return tuple(
          jnp.maximum(accs[g], rows[s][r, pl.ds(g * 16, 16)])
          for g in range(_G))

    accs = lax.fori_loop(0, _L, red, (neg,) * _G, unroll=8)

    for c in range(_C):
      p = accs[0] * w_regs[c][0]
      for g in range(1, _G):
        p = p + accs[g] * w_regs[c][g]
      v = jnp.where(lane == (_C * j + c), jnp.sum(p), v)
    return v

  issue(0, 0)
  issue(1, 1)

  @pl.loop(0, _NB, step=8)
  def _(g):
    v = zero
    for j in range(8):
      b = g + j
      s = j % 2
      wait(s)
      v = compute(s, j, v)
      b2 = b + 2

      @pl.when(b2 < _NB)
      def _():
        issue(b2, s)

    out_v[pl.ds(pl.multiple_of(g * _C, 16), 16)] = v + bp_reg

  pltpu.sync_copy(out_v, out_h.at[pl.ds(base_b * _C, _NB * _C)])


def kernel(x, emb_table, W, b):
  xflat = x.reshape(-1).astype(jnp.int32)
  btiled = jnp.tile(b.astype(jnp.float32), 16 // _C)

  run = pl.kernel(
      _body,
      out_type=jax.ShapeDtypeStruct((_B * _C,), jnp.float32),
      mesh=plsc.VectorSubcoreMesh(core_axis_name="c", subcore_axis_name="s"),
      compiler_params=pltpu.CompilerParams(
          needs_layout_passes=False, use_tc_tiling_on_sc=False),
      scratch_types=[
          pltpu.VMEM((_NB * _L,), jnp.int32),
          pltpu.VMEM((2, _L, _D), jnp.float32),
          pltpu.VMEM((_NB * _C,), jnp.float32),
          pltpu.VMEM((_C, _D), jnp.float32),
          pltpu.VMEM((16,), jnp.float32),
          pltpu.SemaphoreType.DMA,
          pltpu.SemaphoreType.DMA,
      ],
  )
  return run(emb_table, xflat, W, btiled).reshape(_B, _C)


# speedup vs baseline: 3.6169x; 1.1635x over previous
import functools

import jax
import jax.numpy as jnp
from jax import lax
from jax.experimental import pallas as pl
from jax.experimental.pallas import tpu as pltpu
from jax.experimental.pallas import tpu_sc as plsc

_B = 16384
_L = 200
_D = 64
_C = 2

_NC = 2
_NS = 16
_NW = _NC * _NS
_NB = _B // _NW
_G = _D // 16
_SPLIT = 128
_REST = _L - _SPLIT
_GB = 8
_NGRP = _NB // _GB
_NIB = 4


def _body(table_h, xflat_h, w_h, bp_h, out_h,
          idx_v, rows_v, out_v, w_v, b_v, row_sems, idx_sems):
  wid = lax.axis_index("s") * _NC + lax.axis_index("c")
  base_b = wid * _NB

  pltpu.sync_copy(w_h, w_v)
  pltpu.sync_copy(bp_h, b_v)

  w_regs = [[w_v[c, pl.ds(g * 16, 16)] for g in range(_G)] for c in range(_C)]
  bp_reg = b_v[...]
  lane = lax.iota(jnp.int32, 16)
  zero = jnp.zeros((16,), jnp.float32)
  neg = jnp.full((16,), -jnp.inf, dtype=jnp.float32)

  def stage_idx(grp, q):
    off = pl.multiple_of((base_b + grp * _GB) * _L, 8)
    pltpu.async_copy(xflat_h.at[pl.ds(off, _GB * _L)], idx_v.at[q],
                     idx_sems.at[q])

  def wait_idx(q):
    pltpu.make_async_copy(xflat_h.at[pl.ds(0, _GB * _L)], idx_v.at[q],
                          idx_sems.at[q]).wait()

  def issue(q, j):
    off = j * _L
    pltpu.async_copy(table_h.at[idx_v.at[q, pl.ds(off, _SPLIT)]],
                     rows_v.at[j, pl.ds(0, _SPLIT)], row_sems.at[j])
    pltpu.async_copy(table_h.at[idx_v.at[q, pl.ds(off + _SPLIT, _REST)]],
                     rows_v.at[j, pl.ds(_SPLIT, _REST)], row_sems.at[j])

  def wait_rows(j):
    pltpu.make_async_copy(table_h.at[pl.ds(0, _L)], rows_v.at[j],
                          row_sems.at[j]).wait()

  def compute(j, v):
    def red(r, accs):
      return tuple(
          jnp.maximum(accs[g], rows_v[j, r, pl.ds(g * 16, 16)])
          for g in range(_G))

    accs = lax.fori_loop(0, _L, red, (neg,) * _G, unroll=8)

    for c in range(_C):
      p = accs[0] * w_regs[c][0]
      for g in range(1, _G):
        p = p + accs[g] * w_regs[c][g]
      v = jnp.where(lane == (_C * j + c), jnp.sum(p), v)
    return v

  stage_idx(0, 0)
  stage_idx(1, 1)
  wait_idx(0)
  for j in range(_GB):
    issue(0, j)

  @pl.loop(0, _NGRP, step=_NIB)
  def _(g4):
    for q in range(_NIB):
      g = g4 + q

      @pl.when(g + 1 < _NGRP)
      def _():
        wait_idx((q + 1) % _NIB)

      @pl.when(g + 2 < _NGRP)
      def _():
        stage_idx(g + 2, (q + 2) % _NIB)

      v = zero
      for j in range(_GB):
        wait_rows(j)
        v = compute(j, v)

        @pl.when(g + 1 < _NGRP)
        def _():
          issue((q + 1) % _NIB, j)

      out_v[pl.ds(pl.multiple_of(g * _GB * _C, 16), 16)] = v + bp_reg

  pltpu.sync_copy(out_v, out_h.at[pl.ds(base_b * _C, _NB * _C)])


def kernel(x, emb_table, W, b):
  xflat = x.reshape(-1).astype(jnp.int32)
  btiled = jnp.tile(b.astype(jnp.float32), 16 // _C)

  run = pl.kernel(
      _body,
      out_type=jax.ShapeDtypeStruct((_B * _C,), jnp.float32),
      mesh=plsc.VectorSubcoreMesh(core_axis_name="c", subcore_axis_name="s"),
      compiler_params=pltpu.CompilerParams(
          needs_layout_passes=False, use_tc_tiling_on_sc=False),
      scratch_types=[
          pltpu.VMEM((_NIB, _GB * _L), jnp.int32),
          pltpu.VMEM((_GB, _L, _D), jnp.float32),
          pltpu.VMEM((_NB * _C,), jnp.float32),
          pltpu.VMEM((_C, _D), jnp.float32),
          pltpu.VMEM((16,), jnp.float32),
          pltpu.SemaphoreType.DMA((_GB,)),
          pltpu.SemaphoreType.DMA((_NIB,)),
      ],
  )
  return run(emb_table, xflat, W, btiled).reshape(_B, _C)
